# baseline (speedup 1.0000x reference)
import functools

import jax
import jax.numpy as jnp
from jax import lax
from jax.experimental import pallas as pl
from jax.experimental.pallas import tpu as pltpu
from jax.experimental.pallas import tpu_sc as plsc

N = 10000
E = 320000
D = 128
DE = 16
DEPTH = 3
G = 64
MAX_ATOM = 100
MA_PAD = 104

NC, NS = 2, 16
NW = NC * NS
CH = 128
EH = E // 2
EPW = EH // NW
NP = 10112
SPT = NP // NS

_mesh = plsc.VectorSubcoreMesh(core_axis_name="c", subcore_axis_name="s")


GR = 3
GNCH = 42
GNG = GNCH // GR


@functools.partial(
    pl.kernel,
    out_type=(jax.ShapeDtypeStruct((EH, D), jnp.float32),
              jax.ShapeDtypeStruct((EH, D), jnp.float32)),
    mesh=_mesh,
    scratch_types=[
        pltpu.VMEM((GR, CH), jnp.int32),
        pltpu.VMEM((GR, CH), jnp.int32),
        pltpu.VMEM((GR, CH, D), jnp.float32),
        pltpu.VMEM((GR, CH, D), jnp.float32),
        pltpu.SemaphoreType.DMA, pltpu.SemaphoreType.DMA,
        pltpu.SemaphoreType.DMA, pltpu.SemaphoreType.DMA,
        pltpu.SemaphoreType.DMA, pltpu.SemaphoreType.DMA,
        pltpu.SemaphoreType.DMA, pltpu.SemaphoreType.DMA,
        pltpu.SemaphoreType.DMA,
    ],
)
def _sc_gather(x_hbm, src_hbm, dst_hbm, gs_hbm, gd_hbm,
               si_v, di_v, rs_v, rd_v,
               si0, si1, si2, sg0, sg1, sg2, sw0, sw1, sw2):
    sem_i = (si0, si1, si2)
    sem_g = (sg0, sg1, sg2)
    sem_w = (sw0, sw1, sw2)
    wid = lax.axis_index("s") * NC + lax.axis_index("c")
    base = wid * EPW

    def group(g, carry):
        ilds, offs = [], []
        for b in range(GR):
            off = base + jnp.minimum((g * GR + b) * CH, EPW - CH)
            ia = pltpu.async_copy(src_hbm.at[pl.ds(off, CH)], si_v.at[b],
                                  sem_i[b])
            ib = pltpu.async_copy(dst_hbm.at[pl.ds(off, CH)], di_v.at[b],
                                  sem_i[b])
            ilds.append((ia, ib))
            offs.append(off)
        gds = []
        for b in range(GR):
            ia, ib = ilds[b]
            ia.wait()
            ib.wait()
            da = pltpu.async_copy(x_hbm.at[si_v.at[b]], rs_v.at[b], sem_g[b])
            db = pltpu.async_copy(x_hbm.at[di_v.at[b]], rd_v.at[b], sem_g[b])
            gds.append((da, db))
        wbs = []
        for b in range(GR):
            da, db = gds[b]
            da.wait()
            db.wait()
            wa = pltpu.async_copy(rs_v.at[b], gs_hbm.at[pl.ds(offs[b], CH)],
                                  sem_w[b])
            wb = pltpu.async_copy(rd_v.at[b], gd_hbm.at[pl.ds(offs[b], CH)],
                                  sem_w[b])
            wbs.append((wa, wb))
        for wa, wb in wbs:
            wa.wait()
            wb.wait()
        return carry

    lax.fori_loop(0, GNG, group, 0)


SR = 2
SNCH = EPW // CH
TAIL = EPW - SNCH * CH
SNG = SNCH // SR


@functools.partial(
    pl.kernel,
    out_type=jax.ShapeDtypeStruct((NC * NP, D), jnp.float32),
    mesh=_mesh,
    scratch_types=[
        pltpu.VMEM((SR, CH), jnp.int32),
        pltpu.VMEM((SR, CH, D), jnp.float32),
        pltpu.VMEM((TAIL,), jnp.int32),
        pltpu.VMEM((TAIL, D), jnp.float32),
        pltpu.VMEM_SHARED((NP, D), jnp.float32),
        pltpu.SemaphoreType.DMA, pltpu.SemaphoreType.DMA,
        pltpu.SemaphoreType.DMA, pltpu.SemaphoreType.DMA,
        pltpu.SemaphoreType.DMA,
    ],
)
def _sc_scatter(msga_hbm, msgb_hbm, dsta_hbm, dstb_hbm, zeros_hbm, part_hbm,
                di_v, m_v, dt_v, mt_v, agg_sp,
                sm0, sm1, sa0, sa1, sem_t):
    sem_m = (sm0, sm1)
    sem_a = (sa0, sa1)
    c = lax.axis_index("c")
    s = lax.axis_index("s")
    wid = s * NC + c
    base = wid * EPW
    pltpu.sync_copy(zeros_hbm, agg_sp.at[pl.ds(s * SPT, SPT)])
    plsc.subcore_barrier()

    for msg_hbm, dst_hbm in ((msga_hbm, dsta_hbm), (msgb_hbm, dstb_hbm)):
        def group(g, carry):
            lds = []
            for b in range(SR):
                off = base + (g * SR + b) * CH
                pltpu.sync_copy(dst_hbm.at[pl.ds(off, CH)], di_v.at[b])
                lds.append(pltpu.async_copy(msg_hbm.at[pl.ds(off, CH)],
                                            m_v.at[b], sem_m[b]))
            adds = []
            for b in range(SR):
                lds[b].wait()
                adds.append(pltpu.async_copy(m_v.at[b], agg_sp.at[di_v.at[b]],
                                             sem_a[b], add=True))
            for ad in adds:
                ad.wait()
            return carry

        lax.fori_loop(0, SNG, group, 0)
        off = base + (SNCH - 1) * CH
        pltpu.sync_copy(dst_hbm.at[pl.ds(off, CH)], di_v.at[0])
        pltpu.sync_copy(msg_hbm.at[pl.ds(off, CH)], m_v.at[0])
        pltpu.async_copy(m_v.at[0], agg_sp.at[di_v.at[0]], sem_t,
                         add=True).wait()
        off = base + SNCH * CH
        pltpu.sync_copy(dst_hbm.at[pl.ds(off, TAIL)], dt_v)
        pltpu.sync_copy(msg_hbm.at[pl.ds(off, TAIL)], mt_v)
        pltpu.async_copy(mt_v, agg_sp.at[dt_v], sem_t, add=True).wait()

    plsc.subcore_barrier()
    pltpu.sync_copy(agg_sp.at[pl.ds(s * SPT, SPT)],
                    part_hbm.at[pl.ds(c * NP + s * SPT, SPT)])


BE = 3200
NEB = EH // BE


def _edge_body(gs_ref, gd_ref, ef_ref, ws_ref, wd_ref, we_ref, b_ref, msg_ref):
    t = (jnp.dot(gs_ref[...], ws_ref[...], preferred_element_type=jnp.float32)
         + jnp.dot(gd_ref[...], wd_ref[...], preferred_element_type=jnp.float32)
         + jnp.dot(ef_ref[...], we_ref[...], preferred_element_type=jnp.float32)
         + b_ref[...])
    gate = t[:, :D]
    core = t[:, D:]
    msg_ref[...] = jax.nn.sigmoid(gate) * jnp.tanh(core)


_edge_call = pl.pallas_call(
    _edge_body,
    grid=(NEB,),
    in_specs=[
        pl.BlockSpec((BE, D), lambda i: (i, 0)),
        pl.BlockSpec((BE, D), lambda i: (i, 0)),
        pl.BlockSpec((BE, DE), lambda i: (i, 0)),
        pl.BlockSpec((D, 2 * D), lambda i: (0, 0)),
        pl.BlockSpec((D, 2 * D), lambda i: (0, 0)),
        pl.BlockSpec((DE, 2 * D), lambda i: (0, 0)),
        pl.BlockSpec((1, 2 * D), lambda i: (0, 0)),
    ],
    out_specs=pl.BlockSpec((BE, D), lambda i: (i, 0)),
    out_shape=jax.ShapeDtypeStruct((EH, D), jnp.float32),
)


BN = 2000


def _embed_body(nf_ref, tbl_ref, x_ref):
    nf = nf_ref[...][:, 0]
    oh = (nf[:, None] == lax.broadcasted_iota(jnp.int32, (BN, MA_PAD), 1)
          ).astype(jnp.float32)
    x_ref[...] = jnp.dot(oh, tbl_ref[...], preferred_element_type=jnp.float32,
                         precision=lax.Precision.HIGHEST)


_embed_call = pl.pallas_call(
    _embed_body,
    grid=(N // BN,),
    in_specs=[pl.BlockSpec((BN, 1), lambda i: (i, 0)),
              pl.BlockSpec((MA_PAD, D), lambda i: (0, 0))],
    out_specs=pl.BlockSpec((BN, D), lambda i: (i, 0)),
    out_shape=jax.ShapeDtypeStruct((N, D), jnp.float32),
)


def _node_body(x_ref, p_ref, o_ref):
    o_ref[...] = jax.nn.softplus(x_ref[...] + p_ref[0:N] + p_ref[NP:NP + N])


_node_call = pl.pallas_call(
    _node_body,
    in_specs=[pl.BlockSpec((N, D)), pl.BlockSpec((NC * NP, D))],
    out_specs=pl.BlockSpec((N, D)),
    out_shape=jax.ShapeDtypeStruct((N, D), jnp.float32),
)


def _head_body(x_ref, p_ref, cai_ref, w1_ref, b1_ref, w2_ref, b2_ref, o_ref):
    x3 = jax.nn.softplus(x_ref[...] + p_ref[0:N] + p_ref[NP:NP + N])
    seg = cai_ref[...][:, 0]
    oh = (seg[None, :] == lax.broadcasted_iota(jnp.int32, (G, N), 0)
          ).astype(jnp.float32)
    sums = jnp.dot(oh, x3, preferred_element_type=jnp.float32,
                   precision=lax.Precision.HIGHEST)
    counts = jnp.sum(oh, axis=1, keepdims=True)
    gf = sums / (counts + 1e-8)
    h = jnp.maximum(
        jnp.dot(gf, w1_ref[...], preferred_element_type=jnp.float32)
        + b1_ref[...], 0.0)
    o_ref[...] = jnp.dot(h, w2_ref[...],
                         preferred_element_type=jnp.float32) + b2_ref[...]


_head_call = pl.pallas_call(
    _head_body,
    in_specs=[pl.BlockSpec((N, D)), pl.BlockSpec((NC * NP, D)),
              pl.BlockSpec((N, 1)), pl.BlockSpec((D, 128)),
              pl.BlockSpec((1, 128)), pl.BlockSpec((128, 1)),
              pl.BlockSpec((1, 1))],
    out_specs=pl.BlockSpec((G, 1)),
    out_shape=jax.ShapeDtypeStruct((G, 1), jnp.float32),
)


def kernel(node_fea, edge_index, edge_fea, crystal_atom_idx, embed_table,
           conv_W, conv_b, fc1_W, fc1_b, fc2_W, fc2_b):
    nf2 = node_fea.reshape(N, 1)
    tblp = jnp.pad(embed_table, ((0, MA_PAD - MAX_ATOM), (0, 0)))
    x = _embed_call(nf2, tblp)
    srca = edge_index[0, :EH]
    srcb = edge_index[0, EH:]
    dsta = edge_index[1, :EH]
    dstb = edge_index[1, EH:]
    efa = edge_fea[:EH]
    efb = edge_fea[EH:]
    zeros = jnp.zeros((SPT, D), jnp.float32)
    cai2 = crystal_atom_idx.reshape(N, 1)
    b1 = fc1_b.reshape(1, 128)
    b2 = fc2_b.reshape(1, 1)
    out = None
    for i in range(DEPTH):
        ws = conv_W[i, :D]
        wd = conv_W[i, D:2 * D]
        we = conv_W[i, 2 * D:]
        bb = conv_b[i].reshape(1, 2 * D)
        gsa, gda = _sc_gather(x, srca, dsta)
        gsb, gdb = _sc_gather(x, srcb, dstb)
        msga = _edge_call(gsa, gda, efa, ws, wd, we, bb)
        msgb = _edge_call(gsb, gdb, efb, ws, wd, we, bb)
        part = _sc_scatter(msga, msgb, dsta, dstb, zeros)
        if i < DEPTH - 1:
            x = _node_call(x, part)
        else:
            out = _head_call(x, part, cai2, fc1_W, b1, fc2_W, b2)
    return out


# speedup vs baseline: 1.0063x; 1.0063x over previous
import functools

import jax
import jax.numpy as jnp
from jax import lax
from jax.experimental import pallas as pl
from jax.experimental.pallas import tpu as pltpu
from jax.experimental.pallas import tpu_sc as plsc

N = 10000
E = 320000
D = 128
DE = 16
DEPTH = 3
G = 64
MAX_ATOM = 100
MA_PAD = 104

NC, NS = 2, 16
NW = NC * NS
CH = 128
EH = E // 2
EPW = EH // NW
NP = 10112
SPT = NP // NS

_mesh = plsc.VectorSubcoreMesh(core_axis_name="c", subcore_axis_name="s")


GR = 3
GNCH = 42
GNG = GNCH // GR


@functools.partial(
    pl.kernel,
    out_type=(jax.ShapeDtypeStruct((EH, D), jnp.float32),
              jax.ShapeDtypeStruct((EH, D), jnp.float32)),
    mesh=_mesh,
    scratch_types=[
        pltpu.VMEM((GR, CH), jnp.int32),
        pltpu.VMEM((GR, CH), jnp.int32),
        pltpu.VMEM((GR, CH, D), jnp.float32),
        pltpu.VMEM((GR, CH, D), jnp.float32),
        pltpu.SemaphoreType.DMA, pltpu.SemaphoreType.DMA,
        pltpu.SemaphoreType.DMA, pltpu.SemaphoreType.DMA,
        pltpu.SemaphoreType.DMA, pltpu.SemaphoreType.DMA,
    ],
)
def _sc_gather(x_hbm, src_hbm, dst_hbm, gs_hbm, gd_hbm,
               si_v, di_v, rs_v, rd_v,
               sg0, sg1, sg2, sw0, sw1, sw2):
    sem_g = (sg0, sg1, sg2)
    sem_w = (sw0, sw1, sw2)
    wid = lax.axis_index("s") * NC + lax.axis_index("c")
    base = wid * EPW

    def group(g, carry):
        gds, offs = [], []
        for b in range(GR):
            off = base + jnp.minimum((g * GR + b) * CH, EPW - CH)
            pltpu.sync_copy(src_hbm.at[pl.ds(off, CH)], si_v.at[b])
            pltpu.sync_copy(dst_hbm.at[pl.ds(off, CH)], di_v.at[b])
            da = pltpu.async_copy(x_hbm.at[si_v.at[b]], rs_v.at[b], sem_g[b])
            db = pltpu.async_copy(x_hbm.at[di_v.at[b]], rd_v.at[b], sem_g[b])
            gds.append((da, db))
            offs.append(off)
        wbs = []
        for b in range(GR):
            da, db = gds[b]
            da.wait()
            db.wait()
            wa = pltpu.async_copy(rs_v.at[b], gs_hbm.at[pl.ds(offs[b], CH)],
                                  sem_w[b])
            wb = pltpu.async_copy(rd_v.at[b], gd_hbm.at[pl.ds(offs[b], CH)],
                                  sem_w[b])
            wbs.append((wa, wb))
        for wa, wb in wbs:
            wa.wait()
            wb.wait()
        return carry

    lax.fori_loop(0, GNG, group, 0)


SR = 2
SNCH = EPW // CH
TAIL = EPW - SNCH * CH
SNG = SNCH // SR


@functools.partial(
    pl.kernel,
    out_type=jax.ShapeDtypeStruct((NC * NP, D), jnp.float32),
    mesh=_mesh,
    scratch_types=[
        pltpu.VMEM((SR, CH), jnp.int32),
        pltpu.VMEM((SR, CH, D), jnp.float32),
        pltpu.VMEM((TAIL,), jnp.int32),
        pltpu.VMEM((TAIL, D), jnp.float32),
        pltpu.VMEM_SHARED((NP, D), jnp.float32),
        pltpu.SemaphoreType.DMA, pltpu.SemaphoreType.DMA,
        pltpu.SemaphoreType.DMA, pltpu.SemaphoreType.DMA,
        pltpu.SemaphoreType.DMA,
    ],
)
def _sc_scatter(msga_hbm, msgb_hbm, dsta_hbm, dstb_hbm, zeros_hbm, part_hbm,
                di_v, m_v, dt_v, mt_v, agg_sp,
                sm0, sm1, sa0, sa1, sem_t):
    sem_m = (sm0, sm1)
    sem_a = (sa0, sa1)
    c = lax.axis_index("c")
    s = lax.axis_index("s")
    wid = s * NC + c
    base = wid * EPW
    pltpu.sync_copy(zeros_hbm, agg_sp.at[pl.ds(s * SPT, SPT)])
    plsc.subcore_barrier()

    for msg_hbm, dst_hbm in ((msga_hbm, dsta_hbm), (msgb_hbm, dstb_hbm)):
        def group(g, carry):
            lds = []
            for b in range(SR):
                off = base + (g * SR + b) * CH
                pltpu.sync_copy(dst_hbm.at[pl.ds(off, CH)], di_v.at[b])
                lds.append(pltpu.async_copy(msg_hbm.at[pl.ds(off, CH)],
                                            m_v.at[b], sem_m[b]))
            adds = []
            for b in range(SR):
                lds[b].wait()
                adds.append(pltpu.async_copy(m_v.at[b], agg_sp.at[di_v.at[b]],
                                             sem_a[b], add=True))
            for ad in adds:
                ad.wait()
            return carry

        lax.fori_loop(0, SNG, group, 0)
        off = base + (SNCH - 1) * CH
        pltpu.sync_copy(dst_hbm.at[pl.ds(off, CH)], di_v.at[0])
        pltpu.sync_copy(msg_hbm.at[pl.ds(off, CH)], m_v.at[0])
        pltpu.async_copy(m_v.at[0], agg_sp.at[di_v.at[0]], sem_t,
                         add=True).wait()
        off = base + SNCH * CH
        pltpu.sync_copy(dst_hbm.at[pl.ds(off, TAIL)], dt_v)
        pltpu.sync_copy(msg_hbm.at[pl.ds(off, TAIL)], mt_v)
        pltpu.async_copy(mt_v, agg_sp.at[dt_v], sem_t, add=True).wait()

    plsc.subcore_barrier()
    pltpu.sync_copy(agg_sp.at[pl.ds(s * SPT, SPT)],
                    part_hbm.at[pl.ds(c * NP + s * SPT, SPT)])


BE = 3200
NEB = EH // BE


def _edge_body(gs_ref, gd_ref, ef_ref, ws_ref, wd_ref, we_ref, b_ref, msg_ref):
    t = (jnp.dot(gs_ref[...], ws_ref[...], preferred_element_type=jnp.float32)
         + jnp.dot(gd_ref[...], wd_ref[...], preferred_element_type=jnp.float32)
         + jnp.dot(ef_ref[...], we_ref[...], preferred_element_type=jnp.float32)
         + b_ref[...])
    gate = t[:, :D]
    core = t[:, D:]
    msg_ref[...] = jax.nn.sigmoid(gate) * jnp.tanh(core)


_edge_call = pl.pallas_call(
    _edge_body,
    grid=(NEB,),
    in_specs=[
        pl.BlockSpec((BE, D), lambda i: (i, 0)),
        pl.BlockSpec((BE, D), lambda i: (i, 0)),
        pl.BlockSpec((BE, DE), lambda i: (i, 0)),
        pl.BlockSpec((D, 2 * D), lambda i: (0, 0)),
        pl.BlockSpec((D, 2 * D), lambda i: (0, 0)),
        pl.BlockSpec((DE, 2 * D), lambda i: (0, 0)),
        pl.BlockSpec((1, 2 * D), lambda i: (0, 0)),
    ],
    out_specs=pl.BlockSpec((BE, D), lambda i: (i, 0)),
    out_shape=jax.ShapeDtypeStruct((EH, D), jnp.float32),
)


BN = 2000


def _embed_body(nf_ref, tbl_ref, x_ref):
    nf = nf_ref[...][:, 0]
    oh = (nf[:, None] == lax.broadcasted_iota(jnp.int32, (BN, MA_PAD), 1)
          ).astype(jnp.float32)
    x_ref[...] = jnp.dot(oh, tbl_ref[...], preferred_element_type=jnp.float32,
                         precision=lax.Precision.HIGHEST)


_embed_call = pl.pallas_call(
    _embed_body,
    grid=(N // BN,),
    in_specs=[pl.BlockSpec((BN, 1), lambda i: (i, 0)),
              pl.BlockSpec((MA_PAD, D), lambda i: (0, 0))],
    out_specs=pl.BlockSpec((BN, D), lambda i: (i, 0)),
    out_shape=jax.ShapeDtypeStruct((N, D), jnp.float32),
)


def _node_body(x_ref, p_ref, o_ref):
    o_ref[...] = jax.nn.softplus(x_ref[...] + p_ref[0:N] + p_ref[NP:NP + N])


_node_call = pl.pallas_call(
    _node_body,
    in_specs=[pl.BlockSpec((N, D)), pl.BlockSpec((NC * NP, D))],
    out_specs=pl.BlockSpec((N, D)),
    out_shape=jax.ShapeDtypeStruct((N, D), jnp.float32),
)


def _head_body(x_ref, p_ref, cai_ref, w1_ref, b1_ref, w2_ref, b2_ref, o_ref):
    x3 = jax.nn.softplus(x_ref[...] + p_ref[0:N] + p_ref[NP:NP + N])
    seg = cai_ref[...][:, 0]
    oh = (seg[None, :] == lax.broadcasted_iota(jnp.int32, (G, N), 0)
          ).astype(jnp.float32)
    sums = jnp.dot(oh, x3, preferred_element_type=jnp.float32,
                   precision=lax.Precision.HIGHEST)
    counts = jnp.sum(oh, axis=1, keepdims=True)
    gf = sums / (counts + 1e-8)
    h = jnp.maximum(
        jnp.dot(gf, w1_ref[...], preferred_element_type=jnp.float32)
        + b1_ref[...], 0.0)
    o_ref[...] = jnp.dot(h, w2_ref[...],
                         preferred_element_type=jnp.float32) + b2_ref[...]


_head_call = pl.pallas_call(
    _head_body,
    in_specs=[pl.BlockSpec((N, D)), pl.BlockSpec((NC * NP, D)),
              pl.BlockSpec((N, 1)), pl.BlockSpec((D, 128)),
              pl.BlockSpec((1, 128)), pl.BlockSpec((128, 1)),
              pl.BlockSpec((1, 1))],
    out_specs=pl.BlockSpec((G, 1)),
    out_shape=jax.ShapeDtypeStruct((G, 1), jnp.float32),
)


def kernel(node_fea, edge_index, edge_fea, crystal_atom_idx, embed_table,
           conv_W, conv_b, fc1_W, fc1_b, fc2_W, fc2_b):
    nf2 = node_fea.reshape(N, 1)
    tblp = jnp.pad(embed_table, ((0, MA_PAD - MAX_ATOM), (0, 0)))
    x = _embed_call(nf2, tblp)
    srca = edge_index[0, :EH]
    srcb = edge_index[0, EH:]
    dsta = edge_index[1, :EH]
    dstb = edge_index[1, EH:]
    efa = edge_fea[:EH]
    efb = edge_fea[EH:]
    zeros = jnp.zeros((SPT, D), jnp.float32)
    cai2 = crystal_atom_idx.reshape(N, 1)
    b1 = fc1_b.reshape(1, 128)
    b2 = fc2_b.reshape(1, 1)
    out = None
    for i in range(DEPTH):
        ws = conv_W[i, :D]
        wd = conv_W[i, D:2 * D]
        we = conv_W[i, 2 * D:]
        bb = conv_b[i].reshape(1, 2 * D)
        gsa, gda = _sc_gather(x, srca, dsta)
        gsb, gdb = _sc_gather(x, srcb, dstb)
        msga = _edge_call(gsa, gda, efa, ws, wd, we, bb)
        msgb = _edge_call(gsb, gdb, efb, ws, wd, we, bb)
        part = _sc_scatter(msga, msgb, dsta, dstb, zeros)
        if i < DEPTH - 1:
            x = _node_call(x, part)
        else:
            out = _head_call(x, part, cai2, fc1_W, b1, fc2_W, b2)
    return out
